# Initial kernel scaffold; baseline (speedup 1.0000x reference)
import jax
import jax.numpy as jnp
from jax.experimental import pallas as pl


def kernel(node_features, hidden_states, cell_states, temporal_memory, W_ih, W_hh, b_ih, b_hh, Wq, bq, Wk, bk, Wv, bv, node_indices, memory_ptr):
    raise NotImplementedError("write your pallas kernel here")


# speedup vs baseline: 7.5148x; 7.5148x over previous
import functools

import jax
import jax.numpy as jnp
import numpy as np
from jax import lax
from jax.experimental import pallas as pl

_N = 100000
_F = 64
_M = 10
_B = 16384
_DECAY = 0.9
_BLK = 512


def _attn_body(nf_ref, hp_ref, cp_ref, mem_ref, aux_ref,
               wih_ref, whh_ref, bg_ref, wq_ref, wk_ref, wv_ref,
               bq_ref, bk_ref, bv_ref, out_ref):
    nf = nf_ref[...]
    hp = hp_ref[...]
    cp = cp_ref[...]
    f32 = jnp.float32

    dims = (((1,), (1,)), ((), ()))
    gates = (lax.dot_general(nf, wih_ref[...], dims, preferred_element_type=f32)
             + lax.dot_general(hp, whh_ref[...], dims, preferred_element_type=f32)
             + bg_ref[...])
    gi = gates[:, 0 * _F:1 * _F]
    gf = gates[:, 1 * _F:2 * _F]
    gg = gates[:, 2 * _F:3 * _F]
    go = gates[:, 3 * _F:4 * _F]
    i_g = jax.nn.sigmoid(gi)
    f_g = jax.nn.sigmoid(gf)
    g_g = jnp.tanh(gg)
    o_g = jax.nn.sigmoid(go)
    c_new = f_g * cp + i_g * g_g
    h_new = o_g * jnp.tanh(c_new)

    bq = bq_ref[...]
    bk = bk_ref[...]
    q = lax.dot_general(h_new, wq_ref[...], dims, preferred_element_type=f32) + bq
    qk = lax.dot_general(q, wk_ref[...], (((1,), (0,)), ((), ())),
                         preferred_element_type=f32)
    s0 = jnp.sum(q * bk, axis=1, keepdims=True)

    ptr_col = aux_ref[:, 0:1]
    cnt_col = aux_ref[:, 1:2]
    dec = jnp.exp(cnt_col * np.float32(np.log(_DECAY)))

    inv_sqrt = np.float32(1.0 / np.sqrt(_F))
    mem_eff = []
    scores = []
    for m in range(_M):
        mem_m = mem_ref[:, m * _F:(m + 1) * _F]
        m_f = np.float32(m)
        is_set = ptr_col == m_f
        is_dec = ptr_col == np.float32(m + 1)
        e = jnp.where(is_set, nf, mem_m * jnp.where(is_dec, dec, 1.0))
        mem_eff.append(e)
        scores.append((jnp.sum(e * qk, axis=1, keepdims=True) + s0) * inv_sqrt)

    smax = scores[0]
    for m in range(1, _M):
        smax = jnp.maximum(smax, scores[m])
    ctx = jnp.zeros_like(nf)
    z = jnp.zeros_like(smax)
    for m in range(_M):
        w = jnp.exp(scores[m] - smax)
        z = z + w
        ctx = ctx + w * mem_eff[m]
    ctx = ctx / z
    out_ref[...] = (lax.dot_general(ctx, wv_ref[...], dims,
                                    preferred_element_type=f32) + bv_ref[...])


def _attention_call(nf_eff, h_prev, c_prev, mem, aux,
                    W_ih, W_hh, b_gates, Wq, Wk, Wv, bq, bk, bv,
                    interpret=False):
    nblk = _B // _BLK
    row = lambda i: (i, 0)
    full = lambda i: (0, 0)
    return pl.pallas_call(
        _attn_body,
        grid=(nblk,),
        in_specs=[
            pl.BlockSpec((_BLK, _F), row),
            pl.BlockSpec((_BLK, _F), row),
            pl.BlockSpec((_BLK, _F), row),
            pl.BlockSpec((_BLK, _M * _F), row),
            pl.BlockSpec((_BLK, 128), row),
            pl.BlockSpec((4 * _F, _F), full),
            pl.BlockSpec((4 * _F, _F), full),
            pl.BlockSpec((1, 4 * _F), full),
            pl.BlockSpec((_F, _F), full),
            pl.BlockSpec((_F, _F), full),
            pl.BlockSpec((_F, _F), full),
            pl.BlockSpec((1, _F), full),
            pl.BlockSpec((1, _F), full),
            pl.BlockSpec((1, _F), full),
        ],
        out_specs=pl.BlockSpec((_BLK, _F), row),
        out_shape=jax.ShapeDtypeStruct((_B, _F), jnp.float32),
        interpret=interpret,
    )(nf_eff, h_prev, c_prev, mem, aux,
      W_ih, W_hh, b_gates, Wq, Wk, Wv, bq, bk, bv)


def kernel(node_features, hidden_states, cell_states, temporal_memory,
           W_ih, W_hh, b_ih, b_hh, Wq, bq, Wk, bk, Wv, bv,
           node_indices, memory_ptr, interpret=False):
    idx = node_indices.astype(jnp.int32)
    pos = jnp.zeros((_N,), jnp.int32).at[idx].max(
        jnp.arange(_B, dtype=jnp.int32))
    lb = pos[idx]
    cnt = jnp.zeros((_N,), jnp.float32).at[idx].add(1.0)[idx]
    h_prev = hidden_states[idx]
    c_prev = cell_states[idx]
    mem = temporal_memory.reshape(_N, _M * _F)[idx]
    ptr_b = memory_ptr[idx].astype(jnp.float32)
    nf_eff = node_features[lb]

    aux = jnp.zeros((_B, 128), jnp.float32)
    aux = aux.at[:, 0].set(ptr_b).at[:, 1].set(cnt)

    b_gates = (b_ih + b_hh).reshape(1, 4 * _F)
    return _attention_call(nf_eff, h_prev, c_prev, mem, aux,
                           W_ih, W_hh, b_gates, Wq, Wk, Wv,
                           bq.reshape(1, _F), bk.reshape(1, _F),
                           bv.reshape(1, _F), interpret=interpret)


# speedup vs baseline: 8.6433x; 1.1502x over previous
import functools

import jax
import jax.numpy as jnp
import numpy as np
from jax import lax
from jax.experimental import pallas as pl
from jax.experimental.pallas import tpu as pltpu
from jax.experimental.pallas import tpu_sc as plsc

_N = 100000
_F = 64
_M = 10
_B = 16384
_DECAY = 0.9
_BLK = 512

_NC = 2
_NS = 16
_NW = _NC * _NS
_L = 16

_NPOS = 100096
_NCNT = 102400
_CH = 4
_CROWS = 32

_mesh = plsc.VectorSubcoreMesh(core_axis_name="c", subcore_axis_name="s",
                               num_cores=_NC, num_subcores=_NS)


def _dup_body(idx_hbm, lb_out, cnt_out,
              pos_tab, idx_c, stage_i, stage_f, zbuf, ones, cgath, spmem,
              sem):
    cid = lax.axis_index("c")
    sid = lax.axis_index("s")
    iota = lax.iota(jnp.int32, _L)

    def load_chunk(c):
        pltpu.sync_copy(idx_hbm.at[pl.ds(c * _CROWS, _CROWS)], idx_c)

    @pl.when((cid == 1) & (sid == 0))
    def _pos():
        def init_vreg(k, j, c):
            idx_vr = idx_c[j, pl.ds(k * _L, _L)]
            b_v = (c * 4096 + j * 128 + k * _L) + iota
            plsc.store_scatter(pos_tab, [idx_vr], b_v)

        def init_chunk(c, _):
            load_chunk(c)
            def rowf(j, _):
                def vf(k, _):
                    init_vreg(k, j, c)
                    return 0
                return lax.fori_loop(0, 8, vf, 0)
            return lax.fori_loop(0, _CROWS, rowf, 0)

        lax.fori_loop(0, _CH, init_chunk, 0)

        def fix_round(_tot):
            def chunk(c, acc):
                load_chunk(c)
                def rowf(j, a):
                    def vf(k, a2):
                        idx_vr = idx_c[j, pl.ds(k * _L, _L)]
                        b_v = (c * 4096 + j * 128 + k * _L) + iota
                        cur = plsc.load_gather(pos_tab, [idx_vr])
                        m = b_v > cur
                        plsc.store_scatter(pos_tab, [idx_vr], b_v, mask=m)
                        return a2 + plsc.all_reduce_population_count(m)
                    return lax.fori_loop(0, 8, vf, a)
                return lax.fori_loop(0, _CROWS, rowf, acc)
            acc = lax.fori_loop(0, _CH, chunk, jnp.zeros((_L,), jnp.int32))
            return lax.reduce_max(acc, (0,))

        lax.while_loop(lambda t: t > 0, fix_round, jnp.int32(1))

        def lb_chunk(c, _):
            load_chunk(c)
            def rowf(j, _):
                def vf(k, _):
                    idx_vr = idx_c[j, pl.ds(k * _L, _L)]
                    stage_i[j, pl.ds(k * _L, _L)] = plsc.load_gather(
                        pos_tab, [idx_vr])
                    return 0
                return lax.fori_loop(0, 8, vf, 0)
            lax.fori_loop(0, _CROWS, rowf, 0)
            pltpu.sync_copy(stage_i, lb_out.at[pl.ds(c * _CROWS, _CROWS)])
            return 0

        lax.fori_loop(0, _CH, lb_chunk, 0)

    @pl.when((cid == 0) & (sid == 0))
    def _cnt():
        zero_v = jnp.zeros((_L,), jnp.float32)
        one_v = jnp.ones((_L,), jnp.float32)

        def zf(k, _):
            zbuf[pl.ds(k * _L, _L)] = zero_v
            return 0
        lax.fori_loop(0, 4096 // _L, zf, 0)

        def of(k, _):
            ones[pl.ds(k * _L, _L)] = one_v
            return 0
        lax.fori_loop(0, 128 // _L, of, 0)

        def zcopy(t, _):
            pltpu.sync_copy(zbuf, spmem.at[pl.ds(t * 4096, 4096)])
            return 0
        lax.fori_loop(0, _NCNT // 4096, zcopy, 0)

        def add_chunk(c, _):
            load_chunk(c)
            def rowf(j, _):
                pltpu.sync_copy(ones, spmem.at[idx_c.at[j]], add=True)
                return 0
            return lax.fori_loop(0, _CROWS, rowf, 0)
        lax.fori_loop(0, _CH, add_chunk, 0)

        def back_chunk(c, _):
            load_chunk(c)
            def rowf(j, _):
                pltpu.async_copy(spmem.at[idx_c.at[j]], cgath, sem).wait()
                def vf(k, _):
                    stage_f[j, pl.ds(k * _L, _L)] = cgath[pl.ds(k * _L, _L)]
                    return 0
                return lax.fori_loop(0, 8, vf, 0)
            lax.fori_loop(0, _CROWS, rowf, 0)
            pltpu.sync_copy(stage_f, cnt_out.at[pl.ds(c * _CROWS, _CROWS)])
            return 0
        lax.fori_loop(0, _CH, back_chunk, 0)


@functools.partial(
    pl.kernel,
    out_type=(jax.ShapeDtypeStruct((128, 128), jnp.int32),
              jax.ShapeDtypeStruct((128, 128), jnp.float32)),
    mesh=_mesh,
    scratch_types=[
        pltpu.VMEM((_NPOS,), jnp.int32),
        pltpu.VMEM((_CROWS, 128), jnp.int32),
        pltpu.VMEM((_CROWS, 128), jnp.int32),
        pltpu.VMEM((_CROWS, 128), jnp.float32),
        pltpu.VMEM((4096,), jnp.float32),
        pltpu.VMEM((128,), jnp.float32),
        pltpu.VMEM((128,), jnp.float32),
        pltpu.VMEM_SHARED((_NCNT,), jnp.float32),
        pltpu.SemaphoreType.DMA,
    ],
    compiler_params=pltpu.CompilerParams(needs_layout_passes=False),
)
def _dup_call(idx_hbm, lb_out, cnt_out, *rest):
    _dup_body(idx_hbm, lb_out, cnt_out, *rest)


def _gather_body(hc, tm2, mp, nf, idx_hbm, lb_hbm,
                 hc_out, m_out, p_out, nf_out,
                 idx_c, lb_c, rbuf, mbuf, pbuf, sem):
    cid = lax.axis_index("c")
    sid = lax.axis_index("s")
    wid = sid * _NC + cid
    pltpu.sync_copy(idx_hbm.at[pl.ds(wid * 4, 4)], idx_c)
    pltpu.sync_copy(lb_hbm.at[pl.ds(wid * 4, 4)], lb_c)

    def step(j, _):
        base = wid * 512 + j * 128
        ir = idx_c.at[j]
        pltpu.async_copy(hc.at[ir], rbuf, sem).wait()
        pltpu.sync_copy(rbuf, hc_out.at[pl.ds(base, 128)])
        pltpu.async_copy(mp.at[ir], pbuf, sem).wait()
        pltpu.sync_copy(pbuf, p_out.at[pl.ds(base, 128)])
        pltpu.async_copy(nf.at[lb_c.at[j]], rbuf, sem).wait()
        pltpu.sync_copy(rbuf, nf_out.at[pl.ds(base, 128)])
        pltpu.async_copy(tm2.at[ir], mbuf, sem).wait()
        pltpu.sync_copy(mbuf, m_out.at[pl.ds(base, 128)])
        return 0

    lax.fori_loop(0, 4, step, 0)


@functools.partial(
    pl.kernel,
    out_type=(jax.ShapeDtypeStruct((_B, 2 * _F), jnp.float32),
              jax.ShapeDtypeStruct((_B, _M * _F), jnp.float32),
              jax.ShapeDtypeStruct((_B,), jnp.int32),
              jax.ShapeDtypeStruct((_B, 2 * _F), jnp.float32)),
    mesh=_mesh,
    scratch_types=[
        pltpu.VMEM((4, 128), jnp.int32),
        pltpu.VMEM((4, 128), jnp.int32),
        pltpu.VMEM((128, 2 * _F), jnp.float32),
        pltpu.VMEM((128, _M * _F), jnp.float32),
        pltpu.VMEM((128,), jnp.int32),
        pltpu.SemaphoreType.DMA,
    ],
    compiler_params=pltpu.CompilerParams(needs_layout_passes=False),
)
def _gather_call(*args):
    _gather_body(*args)


def _attn_body(nf_ref, hc_ref, mem_ref, aux_ref,
               wih_ref, whh_ref, bg_ref, wq_ref, wk_ref, wv_ref,
               bq_ref, bk_ref, bv_ref, out_ref):
    nf = nf_ref[:, 0:_F]
    hp = hc_ref[:, 0:_F]
    cp = hc_ref[:, _F:2 * _F]
    f32 = jnp.float32

    dims = (((1,), (1,)), ((), ()))
    gates = (lax.dot_general(nf, wih_ref[...], dims, preferred_element_type=f32)
             + lax.dot_general(hp, whh_ref[...], dims, preferred_element_type=f32)
             + bg_ref[...])
    gi = gates[:, 0 * _F:1 * _F]
    gf = gates[:, 1 * _F:2 * _F]
    gg = gates[:, 2 * _F:3 * _F]
    go = gates[:, 3 * _F:4 * _F]
    i_g = jax.nn.sigmoid(gi)
    f_g = jax.nn.sigmoid(gf)
    g_g = jnp.tanh(gg)
    o_g = jax.nn.sigmoid(go)
    c_new = f_g * cp + i_g * g_g
    h_new = o_g * jnp.tanh(c_new)

    bq = bq_ref[...]
    bk = bk_ref[...]
    q = lax.dot_general(h_new, wq_ref[...], dims, preferred_element_type=f32) + bq
    qk = lax.dot_general(q, wk_ref[...], (((1,), (0,)), ((), ())),
                         preferred_element_type=f32)
    s0 = jnp.sum(q * bk, axis=1, keepdims=True)

    ptr_col = aux_ref[:, 0:1]
    cnt_col = aux_ref[:, 1:2]
    dec = jnp.exp(cnt_col * np.float32(np.log(_DECAY)))

    inv_sqrt = np.float32(1.0 / np.sqrt(_F))
    mem_eff = []
    scores = []
    for m in range(_M):
        mem_m = mem_ref[:, m * _F:(m + 1) * _F]
        is_set = ptr_col == np.float32(m)
        is_dec = ptr_col == np.float32(m + 1)
        e = jnp.where(is_set, nf, mem_m * jnp.where(is_dec, dec, 1.0))
        mem_eff.append(e)
        scores.append((jnp.sum(e * qk, axis=1, keepdims=True) + s0) * inv_sqrt)

    smax = scores[0]
    for m in range(1, _M):
        smax = jnp.maximum(smax, scores[m])
    ctx = jnp.zeros_like(nf)
    z = jnp.zeros_like(smax)
    for m in range(_M):
        w = jnp.exp(scores[m] - smax)
        z = z + w
        ctx = ctx + w * mem_eff[m]
    ctx = ctx / z
    out_ref[...] = (lax.dot_general(ctx, wv_ref[...], dims,
                                    preferred_element_type=f32) + bv_ref[...])


def _attention_call(nf_eff, hc_prev, mem, aux,
                    W_ih, W_hh, b_gates, Wq, Wk, Wv, bq, bk, bv,
                    interpret=False):
    nblk = _B // _BLK
    row = lambda i: (i, 0)
    full = lambda i: (0, 0)
    return pl.pallas_call(
        _attn_body,
        grid=(nblk,),
        in_specs=[
            pl.BlockSpec((_BLK, 2 * _F), row),
            pl.BlockSpec((_BLK, 2 * _F), row),
            pl.BlockSpec((_BLK, _M * _F), row),
            pl.BlockSpec((_BLK, 128), row),
            pl.BlockSpec((4 * _F, _F), full),
            pl.BlockSpec((4 * _F, _F), full),
            pl.BlockSpec((1, 4 * _F), full),
            pl.BlockSpec((_F, _F), full),
            pl.BlockSpec((_F, _F), full),
            pl.BlockSpec((_F, _F), full),
            pl.BlockSpec((1, _F), full),
            pl.BlockSpec((1, _F), full),
            pl.BlockSpec((1, _F), full),
        ],
        out_specs=pl.BlockSpec((_BLK, _F), row),
        out_shape=jax.ShapeDtypeStruct((_B, _F), jnp.float32),
        interpret=interpret,
    )(nf_eff, hc_prev, mem, aux,
      W_ih, W_hh, b_gates, Wq, Wk, Wv, bq, bk, bv)


def kernel(node_features, hidden_states, cell_states, temporal_memory,
           W_ih, W_hh, b_ih, b_hh, Wq, bq, Wk, bk, Wv, bv,
           node_indices, memory_ptr):
    idx2d = node_indices.astype(jnp.int32).reshape(128, 128)
    lb2d, cnt2d = _dup_call(idx2d)
    hc = jnp.concatenate([hidden_states, cell_states], axis=1)
    nf_pad = jnp.pad(node_features, ((0, 0), (0, _F)))
    hc_prev, mem, ptr_b, nf_eff = _gather_call(
        hc, temporal_memory.reshape(_N, _M * _F),
        memory_ptr, nf_pad, idx2d, lb2d)

    aux = jnp.zeros((_B, 128), jnp.float32)
    aux = aux.at[:, 0].set(ptr_b.astype(jnp.float32))
    aux = aux.at[:, 1].set(cnt2d.reshape(_B))

    b_gates = (b_ih + b_hh).reshape(1, 4 * _F)
    return _attention_call(nf_eff, hc_prev, mem, aux,
                           W_ih, W_hh, b_gates, Wq, Wk, Wv,
                           bq.reshape(1, _F), bk.reshape(1, _F),
                           bv.reshape(1, _F))


# speedup vs baseline: 9.5789x; 1.1082x over previous
import functools

import jax
import jax.numpy as jnp
import numpy as np
from jax import lax
from jax.experimental import pallas as pl
from jax.experimental.pallas import tpu as pltpu
from jax.experimental.pallas import tpu_sc as plsc

_N = 100000
_F = 64
_M = 10
_B = 16384
_DECAY = 0.9
_BLK = 512

_NC = 2
_NS = 16
_NW = _NC * _NS
_L = 16

_NPOS = 100096
_NCNT = 102400
_CH = 4
_CROWS = 32

def _mesh():
    return plsc.VectorSubcoreMesh(core_axis_name="c", subcore_axis_name="s",
                                  num_cores=_NC, num_subcores=_NS)


def _dup_body(idx_hbm, lb_out, cnt_out,
              pos_tab, idx_c, stage_i, stage_f, zbuf, ones, cgath, spmem,
              sem):
    cid = lax.axis_index("c")
    sid = lax.axis_index("s")
    iota = lax.iota(jnp.int32, _L)

    def load_chunk(c):
        pltpu.sync_copy(idx_hbm.at[pl.ds(c * _CROWS, _CROWS)], idx_c)

    @pl.when((cid == 1) & (sid == 0))
    def _pos():
        def init_vreg(k, j, c):
            idx_vr = idx_c[j, pl.ds(k * _L, _L)]
            b_v = (c * 4096 + j * 128 + k * _L) + iota
            plsc.store_scatter(pos_tab, [idx_vr], b_v)

        def init_chunk(c, _):
            load_chunk(c)
            def rowf(j, _):
                def vf(k, _):
                    init_vreg(k, j, c)
                    return 0
                return lax.fori_loop(0, 8, vf, 0)
            return lax.fori_loop(0, _CROWS, rowf, 0)

        lax.fori_loop(0, _CH, init_chunk, 0)

        def fix_round(_tot):
            def chunk(c, acc):
                load_chunk(c)
                def rowf(j, a):
                    def vf(k, a2):
                        idx_vr = idx_c[j, pl.ds(k * _L, _L)]
                        b_v = (c * 4096 + j * 128 + k * _L) + iota
                        cur = plsc.load_gather(pos_tab, [idx_vr])
                        m = b_v > cur
                        plsc.store_scatter(pos_tab, [idx_vr], b_v, mask=m)
                        return a2 + plsc.all_reduce_population_count(m)
                    return lax.fori_loop(0, 8, vf, a)
                return lax.fori_loop(0, _CROWS, rowf, acc)
            acc = lax.fori_loop(0, _CH, chunk, jnp.zeros((_L,), jnp.int32))
            return lax.reduce_max(acc, (0,))

        lax.while_loop(lambda t: t > 0, fix_round, jnp.int32(1))

        def lb_chunk(c, _):
            load_chunk(c)
            def rowf(j, _):
                def vf(k, _):
                    idx_vr = idx_c[j, pl.ds(k * _L, _L)]
                    stage_i[j, pl.ds(k * _L, _L)] = plsc.load_gather(
                        pos_tab, [idx_vr])
                    return 0
                return lax.fori_loop(0, 8, vf, 0)
            lax.fori_loop(0, _CROWS, rowf, 0)
            pltpu.sync_copy(stage_i, lb_out.at[pl.ds(c * _CROWS, _CROWS)])
            return 0

        lax.fori_loop(0, _CH, lb_chunk, 0)

    @pl.when((cid == 0) & (sid == 0))
    def _cnt():
        zero_v = jnp.zeros((_L,), jnp.float32)
        one_v = jnp.ones((_L,), jnp.float32)

        def zf(k, _):
            zbuf[pl.ds(k * _L, _L)] = zero_v
            return 0
        lax.fori_loop(0, 4096 // _L, zf, 0)

        def of(k, _):
            ones[pl.ds(k * _L, _L)] = one_v
            return 0
        lax.fori_loop(0, 128 // _L, of, 0)

        def zcopy(t, _):
            pltpu.sync_copy(zbuf, spmem.at[pl.ds(t * 4096, 4096)])
            return 0
        lax.fori_loop(0, _NCNT // 4096, zcopy, 0)

        def add_chunk(c, _):
            load_chunk(c)
            def rowf(j, _):
                pltpu.sync_copy(ones, spmem.at[idx_c.at[j]], add=True)
                return 0
            return lax.fori_loop(0, _CROWS, rowf, 0)
        lax.fori_loop(0, _CH, add_chunk, 0)

        def back_chunk(c, _):
            load_chunk(c)
            def rowf(j, _):
                pltpu.async_copy(spmem.at[idx_c.at[j]], cgath, sem).wait()
                def vf(k, _):
                    stage_f[j, pl.ds(k * _L, _L)] = cgath[pl.ds(k * _L, _L)]
                    return 0
                return lax.fori_loop(0, 8, vf, 0)
            lax.fori_loop(0, _CROWS, rowf, 0)
            pltpu.sync_copy(stage_f, cnt_out.at[pl.ds(c * _CROWS, _CROWS)])
            return 0
        lax.fori_loop(0, _CH, back_chunk, 0)


@functools.cache
def _dup_call():
  return functools.partial(
    pl.kernel,
    out_type=(jax.ShapeDtypeStruct((128, 128), jnp.int32),
              jax.ShapeDtypeStruct((128, 128), jnp.float32)),
    mesh=_mesh(),
    scratch_types=[
        pltpu.VMEM((_NPOS,), jnp.int32),
        pltpu.VMEM((_CROWS, 128), jnp.int32),
        pltpu.VMEM((_CROWS, 128), jnp.int32),
        pltpu.VMEM((_CROWS, 128), jnp.float32),
        pltpu.VMEM((4096,), jnp.float32),
        pltpu.VMEM((128,), jnp.float32),
        pltpu.VMEM((128,), jnp.float32),
        pltpu.VMEM_SHARED((_NCNT,), jnp.float32),
        pltpu.SemaphoreType.DMA,
    ],
    compiler_params=pltpu.CompilerParams(needs_layout_passes=False),
  )(_dup_body)


def _gather_body(hc, tm2, mp, nf, idx_hbm, lb_hbm,
                 hc_out, m_out, p_out, nf_out,
                 idx_c, lb_c, rbuf, mbuf, pbuf, sem):
    cid = lax.axis_index("c")
    sid = lax.axis_index("s")
    wid = sid * _NC + cid
    pltpu.sync_copy(idx_hbm.at[pl.ds(wid * 4, 4)], idx_c)
    pltpu.sync_copy(lb_hbm.at[pl.ds(wid * 4, 4)], lb_c)

    def step(j, _):
        base = wid * 512 + j * 128
        ir = idx_c.at[j]
        pltpu.async_copy(hc.at[ir], rbuf, sem).wait()
        pltpu.sync_copy(rbuf, hc_out.at[pl.ds(base, 128)])
        pltpu.async_copy(mp.at[ir], pbuf, sem).wait()
        pltpu.sync_copy(pbuf, p_out.at[pl.ds(base, 128)])
        pltpu.async_copy(nf.at[lb_c.at[j]], rbuf, sem).wait()
        pltpu.sync_copy(rbuf, nf_out.at[pl.ds(base, 128)])
        pltpu.async_copy(tm2.at[ir], mbuf, sem).wait()
        pltpu.sync_copy(mbuf, m_out.at[pl.ds(base, 128)])
        return 0

    lax.fori_loop(0, 4, step, 0)


@functools.cache
def _gather_call():
  return functools.partial(
    pl.kernel,
    out_type=(jax.ShapeDtypeStruct((_B, 2 * _F), jnp.float32),
              jax.ShapeDtypeStruct((_B, _M * _F), jnp.float32),
              jax.ShapeDtypeStruct((_B,), jnp.int32),
              jax.ShapeDtypeStruct((_B, 2 * _F), jnp.float32)),
    mesh=_mesh(),
    scratch_types=[
        pltpu.VMEM((4, 128), jnp.int32),
        pltpu.VMEM((4, 128), jnp.int32),
        pltpu.VMEM((128, 2 * _F), jnp.float32),
        pltpu.VMEM((128, _M * _F), jnp.float32),
        pltpu.VMEM((128,), jnp.int32),
        pltpu.SemaphoreType.DMA,
    ],
    compiler_params=pltpu.CompilerParams(needs_layout_passes=False),
  )(_gather_body)


def _attn_body(nf_ref, hc_ref, mem_ref, aux_ref,
               wg_ref, bg_ref, wq_ref, wk_ref, bq_ref, wv_ref,
               bv_ref, out_ref):
    nf = nf_ref[:, 0:_F]
    cp = hc_ref[:, _F:2 * _F]
    f32 = jnp.float32
    dims = (((1,), (1,)), ((), ()))

    x = jnp.concatenate([nf, hc_ref[:, 0:_F]], axis=1)
    gates = lax.dot_general(x, wg_ref[...], dims,
                            preferred_element_type=f32) + bg_ref[...]
    gi = gates[:, 0 * _F:1 * _F]
    gf = gates[:, 1 * _F:2 * _F]
    gg = gates[:, 2 * _F:3 * _F]
    go = gates[:, 3 * _F:4 * _F]
    c_new = jax.nn.sigmoid(gf) * cp + jax.nn.sigmoid(gi) * jnp.tanh(gg)
    h_new = jax.nn.sigmoid(go) * jnp.tanh(c_new)

    wqk = lax.dot_general(wq_ref[...], wk_ref[...], (((0,), (0,)), ((), ())),
                          preferred_element_type=f32)
    bqk = lax.dot_general(bq_ref[...], wk_ref[...], (((1,), (0,)), ((), ())),
                          preferred_element_type=f32)
    qk = lax.dot_general(h_new, wqk, (((1,), (0,)), ((), ())),
                         preferred_element_type=f32) + bqk

    qk10 = jnp.concatenate([qk] * _M, axis=1)
    prods = jnp.concatenate([mem_ref[...] * qk10, nf * qk], axis=1)
    r_i = lax.broadcasted_iota(jnp.int32, ((_M + 1) * _F, 128), 0)
    c_i = lax.broadcasted_iota(jnp.int32, ((_M + 1) * _F, 128), 1)
    sel = ((r_i // _F) == c_i).astype(f32)
    raw = lax.dot_general(prods, sel, (((1,), (0,)), ((), ())),
                          preferred_element_type=f32)

    ptr_col = aux_ref[:, 0:1]
    cnt_col = aux_ref[:, 1:2]
    dec = jnp.exp(cnt_col * np.float32(np.log(_DECAY)))
    inv_sqrt = np.float32(1.0 / np.sqrt(_F))

    ci = lax.broadcasted_iota(jnp.int32, (raw.shape[0], 128), 1).astype(f32)
    sel_set = ci == ptr_col
    sel_dec = ci == (ptr_col - 1.0)
    snf = raw[:, _M:_M + 1]
    s = jnp.where(sel_set, snf, raw * jnp.where(sel_dec, dec, 1.0)) * inv_sqrt
    s = jnp.where(ci < np.float32(_M), s, np.float32(-1e30))
    smax = jnp.max(s, axis=1, keepdims=True)
    w = jnp.exp(s - smax)
    z = jnp.sum(w, axis=1, keepdims=True)
    wset = jnp.sum(jnp.where(sel_set, w, 0.0), axis=1, keepdims=True)
    wmem = jnp.where(sel_set, 0.0, w * jnp.where(sel_dec, dec, 1.0))

    ctx = wset * nf
    for m in range(_M):
        ctx = ctx + wmem[:, m:m + 1] * mem_ref[:, m * _F:(m + 1) * _F]
    ctx = ctx / z
    out_ref[...] = (lax.dot_general(ctx, wv_ref[...], dims,
                                    preferred_element_type=f32) + bv_ref[...])


def _attention_call(nf_eff, hc_prev, mem, aux,
                    Wg, b_gates, Wq, Wk, bq, Wv, bv,
                    interpret=False):
    nblk = _B // _BLK
    row = lambda i: (i, 0)
    full = lambda i: (0, 0)
    return pl.pallas_call(
        _attn_body,
        grid=(nblk,),
        in_specs=[
            pl.BlockSpec((_BLK, 2 * _F), row),
            pl.BlockSpec((_BLK, 2 * _F), row),
            pl.BlockSpec((_BLK, _M * _F), row),
            pl.BlockSpec((_BLK, 128), row),
            pl.BlockSpec((4 * _F, 2 * _F), full),
            pl.BlockSpec((1, 4 * _F), full),
            pl.BlockSpec((_F, _F), full),
            pl.BlockSpec((_F, _F), full),
            pl.BlockSpec((1, _F), full),
            pl.BlockSpec((_F, _F), full),
            pl.BlockSpec((1, _F), full),
        ],
        out_specs=pl.BlockSpec((_BLK, _F), row),
        out_shape=jax.ShapeDtypeStruct((_B, _F), jnp.float32),
        interpret=interpret,
    )(nf_eff, hc_prev, mem, aux,
      Wg, b_gates, Wq, Wk, bq, Wv, bv)


def kernel(node_features, hidden_states, cell_states, temporal_memory,
           W_ih, W_hh, b_ih, b_hh, Wq, bq, Wk, bk, Wv, bv,
           node_indices, memory_ptr):
    idx2d = node_indices.astype(jnp.int32).reshape(128, 128)
    lb2d, cnt2d = _dup_call()(idx2d)
    hc = jnp.concatenate([hidden_states, cell_states], axis=1)
    nf_pad = jnp.pad(node_features, ((0, 0), (0, _F)))
    hc_prev, mem, ptr_b, nf_eff = _gather_call()(
        hc, temporal_memory.reshape(_N, _M * _F),
        memory_ptr, nf_pad, idx2d, lb2d)

    aux = jnp.zeros((_B, 128), jnp.float32)
    aux = aux.at[:, 0].set(ptr_b.astype(jnp.float32))
    aux = aux.at[:, 1].set(cnt2d.reshape(_B))

    b_gates = (b_ih + b_hh).reshape(1, 4 * _F)
    Wg = jnp.concatenate([W_ih, W_hh], axis=1)
    return _attention_call(nf_eff, hc_prev, mem, aux,
                           Wg, b_gates, Wq, Wk, bq.reshape(1, _F),
                           Wv, bv.reshape(1, _F))


# speedup vs baseline: 9.7421x; 1.0170x over previous
import functools

import jax
import jax.numpy as jnp
import numpy as np
from jax import lax
from jax.experimental import pallas as pl
from jax.experimental.pallas import tpu as pltpu
from jax.experimental.pallas import tpu_sc as plsc

_N = 100000
_F = 64
_M = 10
_B = 16384
_DECAY = 0.9
_BLK = 512

_NC = 2
_NS = 16
_NW = _NC * _NS
_L = 16

_NPOS = 100096
_NCNT = 102400
_CH = 4
_CROWS = 32

def _mesh():
    return plsc.VectorSubcoreMesh(core_axis_name="c", subcore_axis_name="s",
                                  num_cores=_NC, num_subcores=_NS)


def _dup_body(idx_hbm, lb_out, cnt_out,
              pos_tab, idx_c, stage_i, zbuf, ones, cgath, spmem,
              sem):
    cid = lax.axis_index("c")
    sid = lax.axis_index("s")
    iota = lax.iota(jnp.int32, _L)

    def load_chunk(c):
        pltpu.sync_copy(idx_hbm.at[pl.ds(c * _CROWS, _CROWS)], idx_c)

    @pl.when((cid == 1) & (sid == 0))
    def _pos():
        def init_chunk(c, _):
            load_chunk(c)
            def rowf(j, _):
                for k in range(8):
                    idx_vr = idx_c[j, pl.ds(k * _L, _L)]
                    b_v = (c * 4096 + j * 128 + k * _L) + iota
                    plsc.store_scatter(pos_tab, [idx_vr], b_v)
                return 0
            return lax.fori_loop(0, _CROWS, rowf, 0)

        lax.fori_loop(0, _CH, init_chunk, 0)

        def fix_round(_tot):
            def chunk(c, acc):
                load_chunk(c)
                def rowf(j, a):
                    for k in range(8):
                        idx_vr = idx_c[j, pl.ds(k * _L, _L)]
                        b_v = (c * 4096 + j * 128 + k * _L) + iota
                        cur = plsc.load_gather(pos_tab, [idx_vr])
                        m = b_v > cur
                        plsc.store_scatter(pos_tab, [idx_vr], b_v, mask=m)
                        a = a + plsc.all_reduce_population_count(m)
                    return a
                return lax.fori_loop(0, _CROWS, rowf, acc)
            acc = lax.fori_loop(0, _CH, chunk, jnp.zeros((_L,), jnp.int32))
            return lax.reduce_max(acc, (0,))

        lax.while_loop(lambda t: t > 0, fix_round, jnp.int32(1))

        def lb_chunk(c, _):
            load_chunk(c)
            def rowf(j, _):
                for k in range(8):
                    idx_vr = idx_c[j, pl.ds(k * _L, _L)]
                    stage_i[j, pl.ds(k * _L, _L)] = plsc.load_gather(
                        pos_tab, [idx_vr])
                return 0
            lax.fori_loop(0, _CROWS, rowf, 0)
            pltpu.sync_copy(stage_i, lb_out.at[pl.ds(c * _CROWS, _CROWS)])
            return 0

        lax.fori_loop(0, _CH, lb_chunk, 0)

    @pl.when((cid == 0) & (sid == 0))
    def _cnt():
        zero_v = jnp.zeros((_L,), jnp.float32)
        one_v = jnp.ones((_L,), jnp.float32)

        def zf(k, _):
            zbuf[pl.ds(k * _L, _L)] = zero_v
            return 0
        lax.fori_loop(0, 4096 // _L, zf, 0)

        for k in range(128 // _L):
            ones[pl.ds(k * _L, _L)] = one_v

        zd = [pltpu.async_copy(zbuf, spmem.at[pl.ds(t * 4096, 4096)], sem)
              for t in range(_NCNT // 4096)]
        for d in zd:
            d.wait()

        def add_chunk(c, _):
            load_chunk(c)
            ds = [pltpu.async_copy(ones, spmem.at[idx_c.at[j]], sem,
                                   add=True)
                  for j in range(_CROWS)]
            for d in ds:
                d.wait()
            return 0
        lax.fori_loop(0, _CH, add_chunk, 0)

        def back_chunk(c, _):
            load_chunk(c)
            ds = [pltpu.async_copy(spmem.at[idx_c.at[j]], cgath.at[j], sem)
                  for j in range(_CROWS)]
            for d in ds:
                d.wait()
            pltpu.sync_copy(cgath, cnt_out.at[pl.ds(c * _CROWS, _CROWS)])
            return 0
        lax.fori_loop(0, _CH, back_chunk, 0)


@functools.cache
def _dup_call():
  return functools.partial(
    pl.kernel,
    out_type=(jax.ShapeDtypeStruct((128, 128), jnp.int32),
              jax.ShapeDtypeStruct((128, 128), jnp.float32)),
    mesh=_mesh(),
    scratch_types=[
        pltpu.VMEM((_NPOS,), jnp.int32),
        pltpu.VMEM((_CROWS, 128), jnp.int32),
        pltpu.VMEM((_CROWS, 128), jnp.int32),
        pltpu.VMEM((4096,), jnp.float32),
        pltpu.VMEM((128,), jnp.float32),
        pltpu.VMEM((_CROWS, 128), jnp.float32),
        pltpu.VMEM_SHARED((_NCNT,), jnp.float32),
        pltpu.SemaphoreType.DMA,
    ],
    compiler_params=pltpu.CompilerParams(needs_layout_passes=False),
  )(_dup_body)


def _gather_body(hc, tm2, mp, nf, idx_hbm, lb_hbm,
                 hc_out, m_out, p_out, nf_out,
                 idx_c, lb_c, hcb, mb, nfb, pb, sg0, sg1, sw0, sw1):
    cid = lax.axis_index("c")
    sid = lax.axis_index("s")
    wid = sid * _NC + cid
    pltpu.sync_copy(idx_hbm.at[pl.ds(wid * 4, 4)], idx_c)
    pltpu.sync_copy(lb_hbm.at[pl.ds(wid * 4, 4)], lb_c)
    sg = (sg0, sg1)
    sw = (sw0, sw1)
    nstep = 8

    def fire_gathers(t):
        s = t % 2
        ir = idx_c.at[t // 2, pl.ds((t % 2) * 64, 64)]
        lr = lb_c.at[t // 2, pl.ds((t % 2) * 64, 64)]
        return [
            pltpu.async_copy(hc.at[ir], hcb.at[s], sg[s]),
            pltpu.async_copy(mp.at[ir], pb.at[s], sg[s]),
            pltpu.async_copy(nf.at[lr], nfb.at[s], sg[s]),
            pltpu.async_copy(tm2.at[ir], mb.at[s], sg[s]),
        ]

    def fire_writebacks(t):
        s = t % 2
        base = wid * 512 + t * 64
        return [
            pltpu.async_copy(hcb.at[s], hc_out.at[pl.ds(base, 64)], sw[s]),
            pltpu.async_copy(pb.at[s], p_out.at[pl.ds(base, 64)], sw[s]),
            pltpu.async_copy(nfb.at[s], nf_out.at[pl.ds(base, 64)], sw[s]),
            pltpu.async_copy(mb.at[s], m_out.at[pl.ds(base, 64)], sw[s]),
        ]

    wb_prev = []
    g_cur = fire_gathers(0)
    for t in range(nstep):
        for d in g_cur:
            d.wait()
        wb_cur = fire_writebacks(t)
        if t + 1 < nstep:
            for d in wb_prev:
                d.wait()
            g_cur = fire_gathers(t + 1)
        wb_prev = wb_cur
    for d in wb_prev:
        d.wait()


@functools.cache
def _gather_call():
  return functools.partial(
    pl.kernel,
    out_type=(jax.ShapeDtypeStruct((_B, 2 * _F), jnp.float32),
              jax.ShapeDtypeStruct((_B, _M * _F), jnp.float32),
              jax.ShapeDtypeStruct((_B,), jnp.int32),
              jax.ShapeDtypeStruct((_B, 2 * _F), jnp.float32)),
    mesh=_mesh(),
    scratch_types=[
        pltpu.VMEM((4, 128), jnp.int32),
        pltpu.VMEM((4, 128), jnp.int32),
        pltpu.VMEM((2, 64, 2 * _F), jnp.float32),
        pltpu.VMEM((2, 64, _M * _F), jnp.float32),
        pltpu.VMEM((2, 64, 2 * _F), jnp.float32),
        pltpu.VMEM((2, 64), jnp.int32),
        pltpu.SemaphoreType.DMA,
        pltpu.SemaphoreType.DMA,
        pltpu.SemaphoreType.DMA,
        pltpu.SemaphoreType.DMA,
    ],
    compiler_params=pltpu.CompilerParams(needs_layout_passes=False),
  )(_gather_body)


def _attn_body(nf_ref, hc_ref, mem_ref, aux_ref,
               wg_ref, bg_ref, wq_ref, wk_ref, bq_ref, wv_ref,
               bv_ref, out_ref):
    nf = nf_ref[:, 0:_F]
    cp = hc_ref[:, _F:2 * _F]
    f32 = jnp.float32
    dims = (((1,), (1,)), ((), ()))

    x = jnp.concatenate([nf, hc_ref[:, 0:_F]], axis=1)
    gates = lax.dot_general(x, wg_ref[...], dims,
                            preferred_element_type=f32) + bg_ref[...]
    gi = gates[:, 0 * _F:1 * _F]
    gf = gates[:, 1 * _F:2 * _F]
    gg = gates[:, 2 * _F:3 * _F]
    go = gates[:, 3 * _F:4 * _F]
    c_new = jax.nn.sigmoid(gf) * cp + jax.nn.sigmoid(gi) * jnp.tanh(gg)
    h_new = jax.nn.sigmoid(go) * jnp.tanh(c_new)

    wqk = lax.dot_general(wq_ref[...], wk_ref[...], (((0,), (0,)), ((), ())),
                          preferred_element_type=f32)
    bqk = lax.dot_general(bq_ref[...], wk_ref[...], (((1,), (0,)), ((), ())),
                          preferred_element_type=f32)
    qk = lax.dot_general(h_new, wqk, (((1,), (0,)), ((), ())),
                         preferred_element_type=f32) + bqk

    qk10 = jnp.concatenate([qk] * _M, axis=1)
    prods = jnp.concatenate([mem_ref[...] * qk10, nf * qk], axis=1)
    r_i = lax.broadcasted_iota(jnp.int32, ((_M + 1) * _F, 128), 0)
    c_i = lax.broadcasted_iota(jnp.int32, ((_M + 1) * _F, 128), 1)
    sel = ((r_i // _F) == c_i).astype(f32)
    raw = lax.dot_general(prods, sel, (((1,), (0,)), ((), ())),
                          preferred_element_type=f32)

    ptr_col = aux_ref[:, 0:1]
    cnt_col = aux_ref[:, 1:2]
    dec = jnp.exp(cnt_col * np.float32(np.log(_DECAY)))
    inv_sqrt = np.float32(1.0 / np.sqrt(_F))

    ci = lax.broadcasted_iota(jnp.int32, (raw.shape[0], 128), 1).astype(f32)
    sel_set = ci == ptr_col
    sel_dec = ci == (ptr_col - 1.0)
    snf = raw[:, _M:_M + 1]
    s = jnp.where(sel_set, snf, raw * jnp.where(sel_dec, dec, 1.0)) * inv_sqrt
    s = jnp.where(ci < np.float32(_M), s, np.float32(-1e30))
    smax = jnp.max(s, axis=1, keepdims=True)
    w = jnp.exp(s - smax)
    z = jnp.sum(w, axis=1, keepdims=True)
    wset = jnp.sum(jnp.where(sel_set, w, 0.0), axis=1, keepdims=True)
    wmem = jnp.where(sel_set, 0.0, w * jnp.where(sel_dec, dec, 1.0))

    ctx = wset * nf
    for m in range(_M):
        ctx = ctx + wmem[:, m:m + 1] * mem_ref[:, m * _F:(m + 1) * _F]
    ctx = ctx / z
    out_ref[...] = (lax.dot_general(ctx, wv_ref[...], dims,
                                    preferred_element_type=f32) + bv_ref[...])


def _attention_call(nf_eff, hc_prev, mem, aux,
                    Wg, b_gates, Wq, Wk, bq, Wv, bv,
                    interpret=False):
    nblk = _B // _BLK
    row = lambda i: (i, 0)
    full = lambda i: (0, 0)
    return pl.pallas_call(
        _attn_body,
        grid=(nblk,),
        in_specs=[
            pl.BlockSpec((_BLK, 2 * _F), row),
            pl.BlockSpec((_BLK, 2 * _F), row),
            pl.BlockSpec((_BLK, _M * _F), row),
            pl.BlockSpec((_BLK, 128), row),
            pl.BlockSpec((4 * _F, 2 * _F), full),
            pl.BlockSpec((1, 4 * _F), full),
            pl.BlockSpec((_F, _F), full),
            pl.BlockSpec((_F, _F), full),
            pl.BlockSpec((1, _F), full),
            pl.BlockSpec((_F, _F), full),
            pl.BlockSpec((1, _F), full),
        ],
        out_specs=pl.BlockSpec((_BLK, _F), row),
        out_shape=jax.ShapeDtypeStruct((_B, _F), jnp.float32),
        interpret=interpret,
    )(nf_eff, hc_prev, mem, aux,
      Wg, b_gates, Wq, Wk, bq, Wv, bv)


def kernel(node_features, hidden_states, cell_states, temporal_memory,
           W_ih, W_hh, b_ih, b_hh, Wq, bq, Wk, bk, Wv, bv,
           node_indices, memory_ptr):
    idx2d = node_indices.astype(jnp.int32).reshape(128, 128)
    lb2d, cnt2d = _dup_call()(idx2d)
    hc = jnp.concatenate([hidden_states, cell_states], axis=1)
    nf_pad = jnp.pad(node_features, ((0, 0), (0, _F)))
    hc_prev, mem, ptr_b, nf_eff = _gather_call()(
        hc, temporal_memory.reshape(_N, _M * _F),
        memory_ptr, nf_pad, idx2d, lb2d)

    aux = jnp.zeros((_B, 128), jnp.float32)
    aux = aux.at[:, 0].set(ptr_b.astype(jnp.float32))
    aux = aux.at[:, 1].set(cnt2d.reshape(_B))

    b_gates = (b_ih + b_hh).reshape(1, 4 * _F)
    Wg = jnp.concatenate([W_ih, W_hh], axis=1)
    return _attention_call(nf_eff, hc_prev, mem, aux,
                           Wg, b_gates, Wq, Wk, bq.reshape(1, _F),
                           Wv, bv.reshape(1, _F))
